# Initial kernel scaffold; baseline (speedup 1.0000x reference)
import jax
import jax.numpy as jnp
from jax.experimental import pallas as pl


def kernel(f_seq, xyz, W_flow, b_flow, W_diff, b_diff, W_unc, b_unc, W_gp1, b_gp1, W_gp2, b_gp2, W_gate, b_gate, W_r1, b_r1, bn_gamma, bn_beta, W_r2, b_r2, log_dt):
    raise NotImplementedError("write your pallas kernel here")


# speedup vs baseline: 14.8243x; 14.8243x over previous
import functools

import jax
import jax.numpy as jnp
from jax.experimental import pallas as pl
from jax.experimental.pallas import tpu as pltpu

_K = 16
_TAU = 0.15


def _softplus(x):
    return jnp.maximum(x, 0.0) + jnp.log1p(jnp.exp(-jnp.abs(x)))


def _sigmoid(x):
    return 1.0 / (1.0 + jnp.exp(-x))


def _b2f(x):
    return x.astype(jnp.float32)


def _rb(x):
    return x.astype(jnp.bfloat16).astype(jnp.float32)


def _stage1_body(T, N, ffull_ref, fbfull_ref, fbtile_ref, xyzrow_ref,
                 xyztile_ref, xyzrowb_ref, xyztileb_ref,
                 WpT_ref, bp_ref, WgateT_ref, bgate_ref, Wgp1T_ref, bgp1_ref,
                 Wgp2T_ref, bgp2_ref, Wr1fT_ref, Wr1xT_ref, br1_ref, dt_ref,
                 base_ref, h1_ref, bn_ref, Dsel, Dex, A, M):
    t = pl.program_id(1)
    f = ffull_ref[0]
    ftb = fbtile_ref[0]
    PT = xyzrow_ref[0]
    P = xyztile_ref[0]
    PTb = xyzrowb_ref[0]
    Pb = xyztileb_ref[0]
    dt = dt_ref[0:1, 0:1]

    pw = jnp.dot(ftb, WpT_ref[...], preferred_element_type=jnp.float32)
    flow = pw[:, 0:3] + bp_ref[0:1, 0:3]
    diff_in = pw[:, 3:4] + bp_ref[0:1, 3:4]
    unc_in = pw[:, 4:5] + bp_ref[0:1, 4:5]
    diffusion_eff = _softplus(diff_in) * (1.0 + _sigmoid(unc_in))

    fnorm = jnp.sqrt(jnp.sum(flow * flow, axis=1, keepdims=True))
    v = flow / jnp.maximum(fnorm, 1e-6)

    Dex[...] = jnp.square(PT[0:1, :] - P[:, 0:1])
    for d in range(1, 3):
        Dex[...] = Dex[...] + jnp.square(PT[d:d + 1, :] - P[:, d:d + 1])
    A[...] = (PT[0:1, :] - P[:, 0:1]) * v[:, 0:1]
    for d in range(1, 3):
        A[...] = A[...] + (PT[d:d + 1, :] - P[:, d:d + 1]) * v[:, d:d + 1]
    Dex[...] = jnp.sqrt(Dex[...])
    A[...] = jnp.exp(A[...] / jnp.maximum(Dex[...], 1e-6) * (1.0 / _TAU))

    Dsel[...] = _b2f(PTb[0:1, :]) * _b2f(Pb[:, 0:1])
    for d in range(1, 3):
        Dsel[...] = Dsel[...] + _b2f(PTb[d:d + 1, :]) * _b2f(Pb[:, d:d + 1])
    sq_row = (jnp.square(PT[0:1, :]) + jnp.square(PT[1:2, :])
              + jnp.square(PT[2:3, :]))
    sq_col = (jnp.square(P[:, 0:1]) + jnp.square(P[:, 1:2])
              + jnp.square(P[:, 2:3]))
    Dsel[...] = (sq_col + sq_row) - 2.0 * Dsel[...]
    rows = jax.lax.broadcasted_iota(jnp.int32, (T, N), 0) + t * T
    cols = jax.lax.broadcasted_iota(jnp.int32, (T, N), 1)
    Dsel[...] = jnp.where(rows == cols, 1e30, Dsel[...])
    M[...] = jnp.zeros((T, N), jnp.float32)

    def _extract(_, carry):
        Dv = Dsel[...]
        m = jnp.min(Dv, axis=1, keepdims=True)
        sel = Dv == m
        M[...] = jnp.where(sel, 1.0, M[...])
        Dsel[...] = jnp.where(sel, 1e30, Dv)
        return carry

    jax.lax.fori_loop(0, _K, _extract, jnp.float32(0.0))

    sum_d = jnp.sum(M[...] * Dex[...], axis=1, keepdims=True)
    sum_d2 = jnp.sum(M[...] * jnp.square(Dex[...]), axis=1, keepdims=True)
    nd = sum_d * (1.0 / _K)
    nv = sum_d2 * (1.0 / _K) - nd * nd

    A[...] = A[...] * M[...]
    rs = jnp.sum(A[...], axis=1, keepdims=True)
    h_flow = jnp.dot(A[...], f, preferred_element_type=jnp.float32) / rs
    n_mean = jnp.dot(M[...], f, preferred_element_type=jnp.float32) * (1.0 / _K)

    ft = ffull_ref[0, pl.ds(t * T, T), :]
    diff_term = diffusion_eff * (n_mean - ft)
    adv = h_flow - ft

    fmean = jnp.mean(_b2f(fbfull_ref[0]), axis=0, keepdims=True)
    pwg = (jnp.dot(fmean, _b2f(WpT_ref[...]), preferred_element_type=jnp.float32)
           + bp_ref[...])
    hg = bgp1_ref[...]
    for d in range(3):
        hg = hg + _rb(pwg[0:1, d:d + 1]) * _b2f(Wgp1T_ref[d:d + 1, :])
    hg = jnp.maximum(hg, 0.0)
    fgf = (jnp.dot(hg.astype(jnp.bfloat16), Wgp2T_ref[...],
                   preferred_element_type=jnp.float32) + bgp2_ref[...])
    gate = _sigmoid(jnp.dot(ftb, WgateT_ref[...],
                            preferred_element_type=jnp.float32) + bgate_ref[...])
    adv = adv + gate * fgf

    base_ref[0] = ft + dt * (adv + diff_term)

    h1 = (jnp.dot(ftb, Wr1fT_ref[...], preferred_element_type=jnp.float32)
          + br1_ref[...])
    for d in range(3):
        h1 = h1 + _rb(flow[:, d:d + 1]) * _b2f(Wr1xT_ref[d:d + 1, :])
    h1 = h1 + _rb(nd) * _b2f(Wr1xT_ref[3:4, :]) + _rb(nv) * _b2f(Wr1xT_ref[4:5, :])
    h1_ref[0] = h1

    s1 = jnp.sum(h1, axis=0, keepdims=True)
    s2 = jnp.sum(h1 * h1, axis=0, keepdims=True)

    @pl.when(t == 0)
    def _():
        bn_ref[0, 0:1, :] = s1
        bn_ref[0, 1:2, :] = s2

    @pl.when(t != 0)
    def _():
        bn_ref[0, 0:1, :] = bn_ref[0, 0:1, :] + s1
        bn_ref[0, 1:2, :] = bn_ref[0, 1:2, :] + s2


def _stage2_body(Mtot, h1_ref, base_ref, bn_ref, gamma_ref, beta_ref,
                 Wr2T_ref, br2_ref, dt_ref, out_ref):
    bn = bn_ref[...]
    mu = jnp.sum(bn[:, 0, :], axis=0, keepdims=True) * (1.0 / Mtot)
    exx = jnp.sum(bn[:, 1, :], axis=0, keepdims=True) * (1.0 / Mtot)
    var = exx - mu * mu
    inv = jax.lax.rsqrt(var + 1e-5)
    a = gamma_ref[...] * inv
    c = beta_ref[...] - mu * a
    h1n = h1_ref[0] * a + c
    r = jnp.maximum(h1n, 0.0).astype(jnp.bfloat16)
    reac = jnp.dot(r, Wr2T_ref[...], preferred_element_type=jnp.float32) + br2_ref[...]
    out_ref[0] = base_ref[0] + dt_ref[0:1, 0:1] * reac


def kernel(f_seq, xyz, W_flow, b_flow, W_diff, b_diff, W_unc, b_unc,
           W_gp1, b_gp1, W_gp2, b_gp2, W_gate, b_gate, W_r1, b_r1,
           bn_gamma, bn_beta, W_r2, b_r2, log_dt):
    B, L, N, C = f_seq.shape
    bl = B * L
    T = 256 if N % 256 == 0 else N
    nt = N // T

    f3 = f_seq.reshape(bl, N, C)
    fb = f3.astype(jnp.bfloat16)
    xyz3 = xyz.reshape(bl, N, 3)
    xyzT = jnp.transpose(xyz3, (0, 2, 1))
    xyzb = xyz3.astype(jnp.bfloat16)
    xyzTb = xyzT.astype(jnp.bfloat16)

    bf = jnp.bfloat16
    WpT = jnp.concatenate([W_flow, W_diff, W_unc], axis=0).T.astype(bf)
    bp = jnp.concatenate([b_flow, b_diff, b_unc]).reshape(1, 5)
    dt = jnp.clip(jnp.exp(log_dt), 1e-4, 10.0).reshape(1, 1)

    full = lambda *shape: pl.BlockSpec(shape, lambda c, t: (0,) * len(shape))
    base, h1, bn = pl.pallas_call(
        functools.partial(_stage1_body, T, N),
        grid=(bl, nt),
        in_specs=[
            pl.BlockSpec((1, N, C), lambda c, t: (c, 0, 0)),
            pl.BlockSpec((1, N, C), lambda c, t: (c, 0, 0)),
            pl.BlockSpec((1, T, C), lambda c, t: (c, t, 0)),
            pl.BlockSpec((1, 3, N), lambda c, t: (c, 0, 0)),
            pl.BlockSpec((1, T, 3), lambda c, t: (c, t, 0)),
            pl.BlockSpec((1, 3, N), lambda c, t: (c, 0, 0)),
            pl.BlockSpec((1, T, 3), lambda c, t: (c, t, 0)),
            full(C, 5), full(1, 5), full(C, C), full(1, C),
            full(3, C), full(1, C), full(C, C), full(1, C),
            full(C, C), full(5, C), full(1, C), full(1, 1),
        ],
        out_specs=[
            pl.BlockSpec((1, T, C), lambda c, t: (c, t, 0)),
            pl.BlockSpec((1, T, C), lambda c, t: (c, t, 0)),
            pl.BlockSpec((1, 8, C), lambda c, t: (c, 0, 0)),
        ],
        out_shape=[
            jax.ShapeDtypeStruct((bl, N, C), jnp.float32),
            jax.ShapeDtypeStruct((bl, N, C), jnp.float32),
            jax.ShapeDtypeStruct((bl, 8, C), jnp.float32),
        ],
        scratch_shapes=[pltpu.VMEM((T, N), jnp.float32)] * 4,
    )(f3, fb, fb, xyzT, xyz3, xyzTb, xyzb,
      WpT, bp, W_gate.T.astype(bf), b_gate.reshape(1, C),
      W_gp1.T.astype(bf), b_gp1.reshape(1, C),
      W_gp2.T.astype(bf), b_gp2.reshape(1, C),
      W_r1[:, :C].T.astype(bf), W_r1[:, C:].T.astype(bf),
      b_r1.reshape(1, C), dt)

    out = pl.pallas_call(
        functools.partial(_stage2_body, float(bl * N)),
        grid=(bl,),
        in_specs=[
            pl.BlockSpec((1, N, C), lambda c: (c, 0, 0)),
            pl.BlockSpec((1, N, C), lambda c: (c, 0, 0)),
            pl.BlockSpec((bl, 8, C), lambda c: (0, 0, 0)),
            pl.BlockSpec((1, C), lambda c: (0, 0)),
            pl.BlockSpec((1, C), lambda c: (0, 0)),
            pl.BlockSpec((C, C), lambda c: (0, 0)),
            pl.BlockSpec((1, C), lambda c: (0, 0)),
            pl.BlockSpec((1, 1), lambda c: (0, 0)),
        ],
        out_specs=pl.BlockSpec((1, N, C), lambda c: (c, 0, 0)),
        out_shape=jax.ShapeDtypeStruct((bl, N, C), jnp.float32),
    )(h1, base, bn, bn_gamma.reshape(1, C), bn_beta.reshape(1, C),
      W_r2.T.astype(bf), b_r2.reshape(1, C), dt)

    return out.reshape(B, L, N, C)


# speedup vs baseline: 26.7228x; 1.8026x over previous
import functools

import jax
import jax.numpy as jnp
from jax.experimental import pallas as pl
from jax.experimental.pallas import tpu as pltpu

_K = 16
_TAU = 0.15


def _softplus(x):
    return jnp.maximum(x, 0.0) + jnp.log1p(jnp.exp(-jnp.abs(x)))


def _sigmoid(x):
    return 1.0 / (1.0 + jnp.exp(-x))


def _b2f(x):
    return x.astype(jnp.float32)


def _rb(x):
    return x.astype(jnp.bfloat16).astype(jnp.float32)


def _stage1_body(T, N, ftile32_ref, fbfull_ref, fbtile_ref, xyzrow_ref,
                 xyztile_ref, xyzrowb_ref, xyztileb_ref,
                 WpT_ref, bp_ref, WgateT_ref, bgate_ref, Wgp1T_ref, bgp1_ref,
                 Wgp2T_ref, bgp2_ref, Wr1fT_ref, Wr1xT_ref, br1_ref, dt_ref,
                 base_ref, h1_ref, bn_ref, Dsel, Dex, A, M):
    t = pl.program_id(1)
    ftb = fbtile_ref[0]
    PT = xyzrow_ref[0]
    P = xyztile_ref[0]
    PTb = xyzrowb_ref[0]
    Pb = xyztileb_ref[0]
    dt = dt_ref[0:1, 0:1]

    pw = jnp.dot(ftb, WpT_ref[...], preferred_element_type=jnp.float32)
    flow = pw[:, 0:3] + bp_ref[0:1, 0:3]
    diff_in = pw[:, 3:4] + bp_ref[0:1, 3:4]
    unc_in = pw[:, 4:5] + bp_ref[0:1, 4:5]
    diffusion_eff = _softplus(diff_in) * (1.0 + _sigmoid(unc_in))

    fnorm = jnp.sqrt(jnp.sum(flow * flow, axis=1, keepdims=True))
    v = flow / jnp.maximum(fnorm, 1e-6)

    sq_row = jnp.sum(jnp.square(PT), axis=0, keepdims=True)
    sq_col = jnp.sum(jnp.square(P), axis=1, keepdims=True)

    gx = jnp.dot(P, PT, preferred_element_type=jnp.float32)
    dist = jnp.sqrt(jnp.maximum((sq_col + sq_row) - 2.0 * gx, 0.0))
    Dex[...] = dist
    numer = (jnp.dot(v, PT, preferred_element_type=jnp.float32)
             - jnp.sum(v * P, axis=1, keepdims=True))
    A[...] = jnp.exp(numer / jnp.maximum(dist, 1e-6) * (1.0 / _TAU))

    gb = jnp.dot(Pb, PTb, preferred_element_type=jnp.float32)
    rows = jax.lax.broadcasted_iota(jnp.int32, (T, N), 0) + t * T
    cols = jax.lax.broadcasted_iota(jnp.int32, (T, N), 1)
    Dsel[...] = jnp.where(rows == cols, 2e30, (sq_col + sq_row) - 2.0 * gb)

    def _extract(_, carry):
        Dv = Dsel[...]
        m = jnp.min(Dv, axis=1, keepdims=True)
        Dsel[...] = jnp.where(Dv == m, 1e30, Dv)
        return carry

    jax.lax.fori_loop(0, _K, _extract, jnp.float32(0.0))
    M[...] = (Dsel[...] == 1e30).astype(jnp.float32)

    sum_d = jnp.sum(M[...] * Dex[...], axis=1, keepdims=True)
    sum_d2 = jnp.sum(M[...] * jnp.square(Dex[...]), axis=1, keepdims=True)
    nd = sum_d * (1.0 / _K)
    nv = sum_d2 * (1.0 / _K) - nd * nd

    A[...] = A[...] * M[...]
    rs = jnp.sum(A[...], axis=1, keepdims=True)
    fb_all = fbfull_ref[0]
    h_flow = jnp.dot(A[...].astype(jnp.bfloat16), fb_all,
                     preferred_element_type=jnp.float32) / rs
    n_mean = jnp.dot(M[...].astype(jnp.bfloat16), fb_all,
                     preferred_element_type=jnp.float32) * (1.0 / _K)

    ft = ftile32_ref[0]
    diff_term = diffusion_eff * (n_mean - ft)
    adv = h_flow - ft

    fmean = jnp.mean(_b2f(fbfull_ref[0]), axis=0, keepdims=True)
    pwg = (jnp.dot(fmean, _b2f(WpT_ref[...]), preferred_element_type=jnp.float32)
           + bp_ref[...])
    hg = bgp1_ref[...]
    for d in range(3):
        hg = hg + _rb(pwg[0:1, d:d + 1]) * _b2f(Wgp1T_ref[d:d + 1, :])
    hg = jnp.maximum(hg, 0.0)
    fgf = (jnp.dot(hg.astype(jnp.bfloat16), Wgp2T_ref[...],
                   preferred_element_type=jnp.float32) + bgp2_ref[...])
    gate = _sigmoid(jnp.dot(ftb, WgateT_ref[...],
                            preferred_element_type=jnp.float32) + bgate_ref[...])
    adv = adv + gate * fgf

    base_ref[0] = ft + dt * (adv + diff_term)

    h1 = (jnp.dot(ftb, Wr1fT_ref[...], preferred_element_type=jnp.float32)
          + br1_ref[...])
    for d in range(3):
        h1 = h1 + _rb(flow[:, d:d + 1]) * _b2f(Wr1xT_ref[d:d + 1, :])
    h1 = h1 + _rb(nd) * _b2f(Wr1xT_ref[3:4, :]) + _rb(nv) * _b2f(Wr1xT_ref[4:5, :])
    h1_ref[0] = h1

    s1 = jnp.sum(h1, axis=0, keepdims=True)
    s2 = jnp.sum(h1 * h1, axis=0, keepdims=True)

    @pl.when(t == 0)
    def _():
        bn_ref[0, 0:1, :] = s1
        bn_ref[0, 1:2, :] = s2

    @pl.when(t != 0)
    def _():
        bn_ref[0, 0:1, :] = bn_ref[0, 0:1, :] + s1
        bn_ref[0, 1:2, :] = bn_ref[0, 1:2, :] + s2


def _stage2_body(Mtot, h1_ref, base_ref, bn_ref, gamma_ref, beta_ref,
                 Wr2T_ref, br2_ref, dt_ref, out_ref):
    bn = bn_ref[...]
    mu = jnp.sum(bn[:, 0, :], axis=0, keepdims=True) * (1.0 / Mtot)
    exx = jnp.sum(bn[:, 1, :], axis=0, keepdims=True) * (1.0 / Mtot)
    var = exx - mu * mu
    inv = jax.lax.rsqrt(var + 1e-5)
    a = gamma_ref[...] * inv
    c = beta_ref[...] - mu * a
    h1n = h1_ref[0] * a + c
    r = jnp.maximum(h1n, 0.0).astype(jnp.bfloat16)
    reac = jnp.dot(r, Wr2T_ref[...], preferred_element_type=jnp.float32) + br2_ref[...]
    out_ref[0] = base_ref[0] + dt_ref[0:1, 0:1] * reac


def kernel(f_seq, xyz, W_flow, b_flow, W_diff, b_diff, W_unc, b_unc,
           W_gp1, b_gp1, W_gp2, b_gp2, W_gate, b_gate, W_r1, b_r1,
           bn_gamma, bn_beta, W_r2, b_r2, log_dt):
    B, L, N, C = f_seq.shape
    bl = B * L
    T = 256 if N % 256 == 0 else N
    nt = N // T

    f3 = f_seq.reshape(bl, N, C)
    fb = f3.astype(jnp.bfloat16)
    xyz3 = xyz.reshape(bl, N, 3)
    xyzT = jnp.transpose(xyz3, (0, 2, 1))
    xyzb = xyz3.astype(jnp.bfloat16)
    xyzTb = xyzT.astype(jnp.bfloat16)

    bf = jnp.bfloat16
    WpT = jnp.concatenate([W_flow, W_diff, W_unc], axis=0).T.astype(bf)
    bp = jnp.concatenate([b_flow, b_diff, b_unc]).reshape(1, 5)
    dt = jnp.clip(jnp.exp(log_dt), 1e-4, 10.0).reshape(1, 1)

    full = lambda *shape: pl.BlockSpec(shape, lambda c, t: (0,) * len(shape))
    base, h1, bn = pl.pallas_call(
        functools.partial(_stage1_body, T, N),
        grid=(bl, nt),
        in_specs=[
            pl.BlockSpec((1, T, C), lambda c, t: (c, t, 0)),
            pl.BlockSpec((1, N, C), lambda c, t: (c, 0, 0)),
            pl.BlockSpec((1, T, C), lambda c, t: (c, t, 0)),
            pl.BlockSpec((1, 3, N), lambda c, t: (c, 0, 0)),
            pl.BlockSpec((1, T, 3), lambda c, t: (c, t, 0)),
            pl.BlockSpec((1, 3, N), lambda c, t: (c, 0, 0)),
            pl.BlockSpec((1, T, 3), lambda c, t: (c, t, 0)),
            full(C, 5), full(1, 5), full(C, C), full(1, C),
            full(3, C), full(1, C), full(C, C), full(1, C),
            full(C, C), full(5, C), full(1, C), full(1, 1),
        ],
        out_specs=[
            pl.BlockSpec((1, T, C), lambda c, t: (c, t, 0)),
            pl.BlockSpec((1, T, C), lambda c, t: (c, t, 0)),
            pl.BlockSpec((1, 8, C), lambda c, t: (c, 0, 0)),
        ],
        out_shape=[
            jax.ShapeDtypeStruct((bl, N, C), jnp.float32),
            jax.ShapeDtypeStruct((bl, N, C), jnp.float32),
            jax.ShapeDtypeStruct((bl, 8, C), jnp.float32),
        ],
        scratch_shapes=[pltpu.VMEM((T, N), jnp.float32)] * 4,
    )(f3, fb, fb, xyzT, xyz3, xyzTb, xyzb,
      WpT, bp, W_gate.T.astype(bf), b_gate.reshape(1, C),
      W_gp1.T.astype(bf), b_gp1.reshape(1, C),
      W_gp2.T.astype(bf), b_gp2.reshape(1, C),
      W_r1[:, :C].T.astype(bf), W_r1[:, C:].T.astype(bf),
      b_r1.reshape(1, C), dt)

    out = pl.pallas_call(
        functools.partial(_stage2_body, float(bl * N)),
        grid=(bl,),
        in_specs=[
            pl.BlockSpec((1, N, C), lambda c: (c, 0, 0)),
            pl.BlockSpec((1, N, C), lambda c: (c, 0, 0)),
            pl.BlockSpec((bl, 8, C), lambda c: (0, 0, 0)),
            pl.BlockSpec((1, C), lambda c: (0, 0)),
            pl.BlockSpec((1, C), lambda c: (0, 0)),
            pl.BlockSpec((C, C), lambda c: (0, 0)),
            pl.BlockSpec((1, C), lambda c: (0, 0)),
            pl.BlockSpec((1, 1), lambda c: (0, 0)),
        ],
        out_specs=pl.BlockSpec((1, N, C), lambda c: (c, 0, 0)),
        out_shape=jax.ShapeDtypeStruct((bl, N, C), jnp.float32),
    )(h1, base, bn, bn_gamma.reshape(1, C), bn_beta.reshape(1, C),
      W_r2.T.astype(bf), b_r2.reshape(1, C), dt)

    return out.reshape(B, L, N, C)


# speedup vs baseline: 28.7497x; 1.0759x over previous
import functools

import jax
import jax.numpy as jnp
from jax.experimental import pallas as pl
from jax.experimental.pallas import tpu as pltpu

_K = 16
_TAU = 0.15


def _softplus(x):
    return jnp.maximum(x, 0.0) + jnp.log1p(jnp.exp(-jnp.abs(x)))


def _sigmoid(x):
    return 1.0 / (1.0 + jnp.exp(-x))


def _b2f(x):
    return x.astype(jnp.float32)


def _rb(x):
    return x.astype(jnp.bfloat16).astype(jnp.float32)


def _stage1_body(T, N, ftile32_ref, fbfull_ref, fbtile_ref, xyzrow_ref,
                 xyztile_ref, xyzrowb_ref, xyztileb_ref,
                 WpT_ref, bp_ref, WgateT_ref, bgate_ref, Wgp1T_ref, bgp1_ref,
                 Wgp2T_ref, bgp2_ref, Wr1fT_ref, Wr1xT_ref, br1_ref, dt_ref,
                 base_ref, h1_ref, bn_ref, Dsel, A, M):
    t = pl.program_id(1)
    ftb = fbtile_ref[0]
    PT = xyzrow_ref[0]
    P = xyztile_ref[0]
    PTb = xyzrowb_ref[0]
    Pb = xyztileb_ref[0]
    dt = dt_ref[0:1, 0:1]

    pw = jnp.dot(ftb, WpT_ref[...], preferred_element_type=jnp.float32)
    flow = pw[:, 0:3] + bp_ref[0:1, 0:3]
    diff_in = pw[:, 3:4] + bp_ref[0:1, 3:4]
    unc_in = pw[:, 4:5] + bp_ref[0:1, 4:5]
    diffusion_eff = _softplus(diff_in) * (1.0 + _sigmoid(unc_in))

    fnorm = jnp.sqrt(jnp.sum(flow * flow, axis=1, keepdims=True))
    v = flow / jnp.maximum(fnorm, 1e-6)

    sq_row = jnp.sum(jnp.square(PT), axis=0, keepdims=True)
    sq_col = jnp.sum(jnp.square(P), axis=1, keepdims=True)

    gx = jnp.dot(P, PT, preferred_element_type=jnp.float32)
    dist = jnp.sqrt(jnp.maximum((sq_col + sq_row) - 2.0 * gx, 0.0))
    numer = (jnp.dot(v, PT, preferred_element_type=jnp.float32)
             - jnp.sum(v * P, axis=1, keepdims=True))
    A[...] = jnp.exp(numer / jnp.maximum(dist, 1e-6) * (1.0 / _TAU))

    gb = jnp.dot(Pb, PTb, preferred_element_type=jnp.float32)
    rows = jax.lax.broadcasted_iota(jnp.int32, (T, N), 0) + t * T
    cols = jax.lax.broadcasted_iota(jnp.int32, (T, N), 1)
    Dsel[...] = jnp.where(rows == cols, 2e30, (sq_col + sq_row) - 2.0 * gb)

    def _extract(_, carry):
        sum_d, sum_d2 = carry
        for _u in range(2):
            Dv = Dsel[...]
            m = jnp.min(Dv, axis=1, keepdims=True)
            Dsel[...] = jnp.where(Dv == m, 1e30, Dv)
            m = jnp.maximum(m, 0.0)
            sum_d = sum_d + jnp.sqrt(m)
            sum_d2 = sum_d2 + m
        return sum_d, sum_d2

    sum_d, sum_d2 = jax.lax.fori_loop(
        0, _K // 2, _extract,
        (jnp.zeros((T, 1), jnp.float32), jnp.zeros((T, 1), jnp.float32)))
    M[...] = (Dsel[...] == 1e30).astype(jnp.float32)
    nd = sum_d * (1.0 / _K)
    nv = sum_d2 * (1.0 / _K) - nd * nd

    A[...] = A[...] * M[...]
    rs = jnp.sum(A[...], axis=1, keepdims=True)
    fb_all = fbfull_ref[0]
    h_flow = jnp.dot(A[...].astype(jnp.bfloat16), fb_all,
                     preferred_element_type=jnp.float32) / rs
    n_mean = jnp.dot(M[...].astype(jnp.bfloat16), fb_all,
                     preferred_element_type=jnp.float32) * (1.0 / _K)

    ft = ftile32_ref[0]
    diff_term = diffusion_eff * (n_mean - ft)
    adv = h_flow - ft

    fmean = jnp.mean(_b2f(fbfull_ref[0]), axis=0, keepdims=True)
    pwg = (jnp.dot(fmean, _b2f(WpT_ref[...]), preferred_element_type=jnp.float32)
           + bp_ref[...])
    hg = bgp1_ref[...]
    for d in range(3):
        hg = hg + _rb(pwg[0:1, d:d + 1]) * _b2f(Wgp1T_ref[d:d + 1, :])
    hg = jnp.maximum(hg, 0.0)
    fgf = (jnp.dot(hg.astype(jnp.bfloat16), Wgp2T_ref[...],
                   preferred_element_type=jnp.float32) + bgp2_ref[...])
    gate = _sigmoid(jnp.dot(ftb, WgateT_ref[...],
                            preferred_element_type=jnp.float32) + bgate_ref[...])
    adv = adv + gate * fgf

    base_ref[0] = ft + dt * (adv + diff_term)

    h1 = (jnp.dot(ftb, Wr1fT_ref[...], preferred_element_type=jnp.float32)
          + br1_ref[...])
    for d in range(3):
        h1 = h1 + _rb(flow[:, d:d + 1]) * _b2f(Wr1xT_ref[d:d + 1, :])
    h1 = h1 + _rb(nd) * _b2f(Wr1xT_ref[3:4, :]) + _rb(nv) * _b2f(Wr1xT_ref[4:5, :])
    h1_ref[0] = h1

    s1 = jnp.sum(h1, axis=0, keepdims=True)
    s2 = jnp.sum(h1 * h1, axis=0, keepdims=True)

    @pl.when(t == 0)
    def _():
        bn_ref[0, 0:1, :] = s1
        bn_ref[0, 1:2, :] = s2

    @pl.when(t != 0)
    def _():
        bn_ref[0, 0:1, :] = bn_ref[0, 0:1, :] + s1
        bn_ref[0, 1:2, :] = bn_ref[0, 1:2, :] + s2


def _stage2_body(Mtot, h1_ref, base_ref, bn_ref, gamma_ref, beta_ref,
                 Wr2T_ref, br2_ref, dt_ref, out_ref):
    bn = bn_ref[...]
    mu = jnp.sum(bn[:, 0, :], axis=0, keepdims=True) * (1.0 / Mtot)
    exx = jnp.sum(bn[:, 1, :], axis=0, keepdims=True) * (1.0 / Mtot)
    var = exx - mu * mu
    inv = jax.lax.rsqrt(var + 1e-5)
    a = gamma_ref[...] * inv
    c = beta_ref[...] - mu * a
    h1n = h1_ref[0] * a + c
    r = jnp.maximum(h1n, 0.0).astype(jnp.bfloat16)
    reac = jnp.dot(r, Wr2T_ref[...], preferred_element_type=jnp.float32) + br2_ref[...]
    out_ref[0] = base_ref[0] + dt_ref[0:1, 0:1] * reac


def kernel(f_seq, xyz, W_flow, b_flow, W_diff, b_diff, W_unc, b_unc,
           W_gp1, b_gp1, W_gp2, b_gp2, W_gate, b_gate, W_r1, b_r1,
           bn_gamma, bn_beta, W_r2, b_r2, log_dt):
    B, L, N, C = f_seq.shape
    bl = B * L
    T = 512 if N % 512 == 0 else N
    nt = N // T

    f3 = f_seq.reshape(bl, N, C)
    fb = f3.astype(jnp.bfloat16)
    xyz3 = xyz.reshape(bl, N, 3)
    xyzT = jnp.transpose(xyz3, (0, 2, 1))
    xyzb = xyz3.astype(jnp.bfloat16)
    xyzTb = xyzT.astype(jnp.bfloat16)

    bf = jnp.bfloat16
    WpT = jnp.concatenate([W_flow, W_diff, W_unc], axis=0).T.astype(bf)
    bp = jnp.concatenate([b_flow, b_diff, b_unc]).reshape(1, 5)
    dt = jnp.clip(jnp.exp(log_dt), 1e-4, 10.0).reshape(1, 1)

    full = lambda *shape: pl.BlockSpec(shape, lambda c, t: (0,) * len(shape))
    base, h1, bn = pl.pallas_call(
        functools.partial(_stage1_body, T, N),
        grid=(bl, nt),
        in_specs=[
            pl.BlockSpec((1, T, C), lambda c, t: (c, t, 0)),
            pl.BlockSpec((1, N, C), lambda c, t: (c, 0, 0)),
            pl.BlockSpec((1, T, C), lambda c, t: (c, t, 0)),
            pl.BlockSpec((1, 3, N), lambda c, t: (c, 0, 0)),
            pl.BlockSpec((1, T, 3), lambda c, t: (c, t, 0)),
            pl.BlockSpec((1, 3, N), lambda c, t: (c, 0, 0)),
            pl.BlockSpec((1, T, 3), lambda c, t: (c, t, 0)),
            full(C, 5), full(1, 5), full(C, C), full(1, C),
            full(3, C), full(1, C), full(C, C), full(1, C),
            full(C, C), full(5, C), full(1, C), full(1, 1),
        ],
        out_specs=[
            pl.BlockSpec((1, T, C), lambda c, t: (c, t, 0)),
            pl.BlockSpec((1, T, C), lambda c, t: (c, t, 0)),
            pl.BlockSpec((1, 8, C), lambda c, t: (c, 0, 0)),
        ],
        out_shape=[
            jax.ShapeDtypeStruct((bl, N, C), jnp.float32),
            jax.ShapeDtypeStruct((bl, N, C), jnp.float32),
            jax.ShapeDtypeStruct((bl, 8, C), jnp.float32),
        ],
        scratch_shapes=[pltpu.VMEM((T, N), jnp.float32)] * 3,
    )(f3, fb, fb, xyzT, xyz3, xyzTb, xyzb,
      WpT, bp, W_gate.T.astype(bf), b_gate.reshape(1, C),
      W_gp1.T.astype(bf), b_gp1.reshape(1, C),
      W_gp2.T.astype(bf), b_gp2.reshape(1, C),
      W_r1[:, :C].T.astype(bf), W_r1[:, C:].T.astype(bf),
      b_r1.reshape(1, C), dt)

    out = pl.pallas_call(
        functools.partial(_stage2_body, float(bl * N)),
        grid=(bl,),
        in_specs=[
            pl.BlockSpec((1, N, C), lambda c: (c, 0, 0)),
            pl.BlockSpec((1, N, C), lambda c: (c, 0, 0)),
            pl.BlockSpec((bl, 8, C), lambda c: (0, 0, 0)),
            pl.BlockSpec((1, C), lambda c: (0, 0)),
            pl.BlockSpec((1, C), lambda c: (0, 0)),
            pl.BlockSpec((C, C), lambda c: (0, 0)),
            pl.BlockSpec((1, C), lambda c: (0, 0)),
            pl.BlockSpec((1, 1), lambda c: (0, 0)),
        ],
        out_specs=pl.BlockSpec((1, N, C), lambda c: (c, 0, 0)),
        out_shape=jax.ShapeDtypeStruct((bl, N, C), jnp.float32),
    )(h1, base, bn, bn_gamma.reshape(1, C), bn_beta.reshape(1, C),
      W_r2.T.astype(bf), b_r2.reshape(1, C), dt)

    return out.reshape(B, L, N, C)
